# Initial kernel scaffold; baseline (speedup 1.0000x reference)
import jax
import jax.numpy as jnp
from jax.experimental import pallas as pl


def kernel(X, mask_raw):
    raise NotImplementedError("write your pallas kernel here")


# speedup vs baseline: 810.3062x; 810.3062x over previous
import functools

import jax
import jax.numpy as jnp
from jax import lax
from jax.experimental import pallas as pl
from jax.experimental.pallas import tpu as pltpu
from jax.experimental.pallas import tpu_sc as plsc

_K = 5
_TOL = 1e-4
_N = 2048 * 32
_NC, _NS, _L = 2, 16, 16
_NW = _NC * _NS
_CHUNK = _N // _NW
_PAD = 8
_NP = _N + 2 * _PAD
_BIG = 3.4028235e38


def _impute_pass(s_ref, q_ref, m_ref, out_ref):
    def body(i, dacc):
        q = q_ref[pl.ds(i * _L, _L)]
        m = m_ref[pl.ds(i * _L, _L)]
        pos = jnp.zeros((_L,), jnp.int32)
        for k in range(16, -1, -1):
            step = 1 << k
            cand = jnp.minimum(pos + step, _N)
            sv = plsc.load_gather(s_ref, [cand + (_PAD - 1)])
            pos = jnp.where(sv < q, cand, pos)
        lidx = pos + (_PAD - 1)
        ridx = pos + _PAD
        acc = jnp.zeros((_L,), jnp.float32)
        for _ in range(_K):
            sl = plsc.load_gather(s_ref, [lidx])
            sr = plsc.load_gather(s_ref, [ridx])
            take_l = (q - sl) <= (sr - q)
            acc = acc + jnp.where(take_l, sl, sr)
            lidx = jnp.where(take_l, lidx - 1, lidx)
            ridx = jnp.where(take_l, ridx, ridx + 1)
        imp = acc / jnp.float32(_K)
        xn = jnp.where(m > 0, q, imp)
        out_ref[pl.ds(i * _L, _L)] = xn
        return dacc + jnp.abs(xn - q)

    return lax.fori_loop(0, _CHUNK // _L, body, jnp.zeros((_L,), jnp.float32))


@functools.partial(
    pl.kernel,
    mesh=plsc.VectorSubcoreMesh(
        core_axis_name="c", subcore_axis_name="s", num_cores=_NC, num_subcores=_NS
    ),
    out_type=(
        jax.ShapeDtypeStruct((_N,), jnp.float32),
        jax.ShapeDtypeStruct((_N,), jnp.float32),
        jax.ShapeDtypeStruct((_NW, _L), jnp.float32),
    ),
    scratch_types=[
        pltpu.VMEM((_NP,), jnp.float32),
        pltpu.VMEM((_CHUNK,), jnp.float32),
        pltpu.VMEM((_CHUNK,), jnp.int32),
        pltpu.VMEM((_CHUNK,), jnp.float32),
        pltpu.VMEM((_CHUNK,), jnp.float32),
        pltpu.VMEM((_L,), jnp.float32),
    ],
    compiler_params=pltpu.CompilerParams(needs_layout_passes=False),
)
def _sc_impute(s_hbm, x_hbm, m_hbm, o1_hbm, o2_hbm, d_hbm,
               s_v, x_v, m_v, o1_v, o2_v, d_v):
    wid = lax.axis_index("s") * _NC + lax.axis_index("c")
    base = wid * _CHUNK
    pltpu.sync_copy(s_hbm, s_v)
    pltpu.sync_copy(x_hbm.at[pl.ds(base, _CHUNK)], x_v)
    pltpu.sync_copy(m_hbm.at[pl.ds(base, _CHUNK)], m_v)
    d1 = _impute_pass(s_v, x_v, m_v, o1_v)
    _impute_pass(s_v, o1_v, m_v, o2_v)
    d_v[...] = d1
    pltpu.sync_copy(o1_v, o1_hbm.at[pl.ds(base, _CHUNK)])
    pltpu.sync_copy(o2_v, o2_hbm.at[pl.ds(base, _CHUNK)])
    pltpu.sync_copy(d_v, d_hbm.at[wid])


def kernel(X, mask_raw):
    obs = (mask_raw > 0).reshape(-1)
    flat = X.reshape(-1)
    pool = jnp.sort(jnp.where(obs, flat, jnp.float32(_BIG)))
    pool_padded = jnp.concatenate([
        jnp.full((_PAD,), -_BIG, jnp.float32),
        pool,
        jnp.full((_PAD,), _BIG, jnp.float32),
    ])
    xn1, xn2, dparts = _sc_impute(pool_padded, flat, obs.astype(jnp.int32))
    diff1 = jnp.sum(dparts)
    out = jnp.where(diff1 < _TOL, xn1, xn2)
    return out.reshape(X.shape)
